# initial kernel scaffold (unmeasured)
import jax
import jax.numpy as jnp
from jax import lax
from jax.experimental import pallas as pl
from jax.experimental.pallas import tpu as pltpu

N_DEV = 32
M_PER = 256
K = 8192
N_TOT = 4096
N_PER = 128


def _gemm_relu_body(x_ref, w_ref, y_ref):
    y_ref[...] = jnp.maximum(
        jnp.dot(x_ref[...], w_ref[...], preferred_element_type=jnp.float32),
        0.0,
    )


def _a2a_body(y_ref, out_ref, send_sems, recv_sems):
    my = lax.axis_index("i")

    sends = []
    for s in range(1, N_DEV):
        dst = lax.rem(my + s, N_DEV)
        rdma = pltpu.make_async_remote_copy(
            src_ref=y_ref.at[:, pl.ds(dst * N_PER, N_PER)],
            dst_ref=out_ref.at[pl.ds(my * M_PER, M_PER), :],
            send_sem=send_sems.at[dst],
            recv_sem=recv_sems.at[my],
            device_id=(dst,),
            device_id_type=pl.DeviceIdType.MESH,
        )
        rdma.start()
        sends.append(rdma)

    out_ref[pl.ds(my * M_PER, M_PER), :] = y_ref[:, pl.ds(my * N_PER, N_PER)]

    for s in range(1, N_DEV):
        src = lax.rem(my + N_DEV - s, N_DEV)
        recv = pltpu.make_async_remote_copy(
            src_ref=y_ref.at[:, pl.ds(src * N_PER, N_PER)],
            dst_ref=out_ref.at[pl.ds(src * M_PER, M_PER), :],
            send_sem=send_sems.at[src],
            recv_sem=recv_sems.at[src],
            device_id=(src,),
            device_id_type=pl.DeviceIdType.MESH,
        )
        recv.wait_recv()

    for rdma in sends:
        rdma.wait_send()


def kernel(x, w_mat):
    y_local = pl.pallas_call(
        _gemm_relu_body,
        grid=(N_DEV,),
        in_specs=[
            pl.BlockSpec((M_PER, K), lambda j: (0, 0), memory_space=pltpu.VMEM),
            pl.BlockSpec((K, N_PER), lambda j: (0, j), memory_space=pltpu.VMEM),
        ],
        out_specs=pl.BlockSpec(
            (M_PER, N_PER), lambda j: (0, j), memory_space=pltpu.VMEM
        ),
        out_shape=jax.ShapeDtypeStruct((M_PER, N_TOT), jnp.float32),
    )(x, w_mat)

    return pl.pallas_call(
        _a2a_body,
        in_specs=[pl.BlockSpec(memory_space=pltpu.VMEM)],
        out_specs=pl.BlockSpec(memory_space=pltpu.VMEM),
        out_shape=jax.ShapeDtypeStruct((N_DEV * M_PER, N_PER), jnp.float32),
        scratch_shapes=[
            pltpu.SemaphoreType.DMA((N_DEV,)),
            pltpu.SemaphoreType.DMA((N_DEV,)),
        ],
        compiler_params=pltpu.CompilerParams(collective_id=0),
    )(y_local)


# baseline (device time: 128881 ns/iter reference)
import jax
import jax.numpy as jnp
from jax import lax
from jax.experimental import pallas as pl
from jax.experimental.pallas import tpu as pltpu

N_DEV = 32
M_PER = 256
K = 8192
N_TOT = 4096
N_PER = 128


def _gemm_relu_body(x_ref, w_ref, y_ref):
    y_ref[...] = jnp.maximum(
        jnp.dot(x_ref[...], w_ref[...], preferred_element_type=jnp.float32),
        0.0,
    )


def _a2a_body(y_ref, out_ref, send_sems, recv_sems):
    my = lax.axis_index("i")

    sends = []
    for s in range(1, N_DEV):
        dst = lax.rem(my + s, N_DEV)
        rdma = pltpu.make_async_remote_copy(
            src_ref=y_ref.at[:, pl.ds(dst * N_PER, N_PER)],
            dst_ref=out_ref.at[pl.ds(my * M_PER, M_PER), :],
            send_sem=send_sems.at[dst],
            recv_sem=recv_sems.at[my],
            device_id=(dst,),
            device_id_type=pl.DeviceIdType.MESH,
        )
        rdma.start()
        sends.append(rdma)

    out_ref[pl.ds(my * M_PER, M_PER), :] = y_ref[:, pl.ds(my * N_PER, N_PER)]

    for s in range(1, N_DEV):
        src = lax.rem(my + N_DEV - s, N_DEV)
        recv = pltpu.make_async_remote_copy(
            src_ref=y_ref.at[:, pl.ds(src * N_PER, N_PER)],
            dst_ref=out_ref.at[pl.ds(src * M_PER, M_PER), :],
            send_sem=send_sems.at[src],
            recv_sem=recv_sems.at[src],
            device_id=(src,),
            device_id_type=pl.DeviceIdType.MESH,
        )
        recv.wait_recv()

    for rdma in sends:
        rdma.wait_send()


def kernel(x, w_mat):
    y_local = pl.pallas_call(
        _gemm_relu_body,
        grid=(N_DEV,),
        in_specs=[
            pl.BlockSpec((M_PER, K), lambda j: (0, 0), memory_space=pltpu.VMEM),
            pl.BlockSpec((K, N_PER), lambda j: (0, j), memory_space=pltpu.VMEM),
        ],
        out_specs=pl.BlockSpec(
            (M_PER, N_PER), lambda j: (0, j), memory_space=pltpu.VMEM
        ),
        out_shape=jax.ShapeDtypeStruct((M_PER, N_TOT), jnp.float32),
    )(x, w_mat)

    return pl.pallas_call(
        _a2a_body,
        in_specs=[pl.BlockSpec(memory_space=pltpu.VMEM)],
        out_specs=pl.BlockSpec(memory_space=pltpu.VMEM),
        out_shape=jax.ShapeDtypeStruct((N_DEV * M_PER, N_PER), jnp.float32),
        scratch_shapes=[
            pltpu.SemaphoreType.DMA((N_DEV,)),
            pltpu.SemaphoreType.DMA((N_DEV,)),
        ],
    )(y_local)


# device time: 90792 ns/iter; 1.4195x vs baseline; 1.4195x over previous
import jax
import jax.numpy as jnp
from jax import lax
from jax.experimental import pallas as pl
from jax.experimental.pallas import tpu as pltpu

N_DEV = 32
M_PER = 256
K = 8192
N_TOT = 4096
N_PER = 128
GRP = 4
GW = GRP * N_PER
N_GRP = N_TOT // GW


def _body(sched_ref, x_ref, w_ref, out_ref, y_buf, send_sems, recv_sems):
    t = pl.program_id(0)
    my = lax.axis_index("i")
    g = sched_ref[t]
    slot = lax.rem(t, 2)

    @pl.when(t >= 2)
    def _():
        g_old = sched_ref[t - 2]
        for j in range(GRP):
            dst = g_old * GRP + j

            @pl.when(dst != my)
            def _():
                pltpu.make_async_remote_copy(
                    src_ref=y_buf.at[slot, :, pl.ds(j * N_PER, N_PER)],
                    dst_ref=out_ref.at[pl.ds(my * M_PER, M_PER), :],
                    send_sem=send_sems.at[dst],
                    recv_sem=recv_sems.at[my],
                    device_id=(dst,),
                    device_id_type=pl.DeviceIdType.MESH,
                ).wait_send()

    y_buf[slot] = jnp.maximum(
        jnp.dot(x_ref[...], w_ref[...], preferred_element_type=jnp.float32),
        0.0,
    )

    for j in range(GRP):
        dst = g * GRP + j

        @pl.when(dst != my)
        def _():
            pltpu.make_async_remote_copy(
                src_ref=y_buf.at[slot, :, pl.ds(j * N_PER, N_PER)],
                dst_ref=out_ref.at[pl.ds(my * M_PER, M_PER), :],
                send_sem=send_sems.at[dst],
                recv_sem=recv_sems.at[my],
                device_id=(dst,),
                device_id_type=pl.DeviceIdType.MESH,
            ).start()

        @pl.when(dst == my)
        def _():
            out_ref[pl.ds(my * M_PER, M_PER), :] = y_buf[
                slot, :, pl.ds(j * N_PER, N_PER)
            ]

    @pl.when(t == N_GRP - 1)
    def _():
        for tt in (N_GRP - 2, N_GRP - 1):
            g_late = sched_ref[tt]
            for j in range(GRP):
                dst = g_late * GRP + j

                @pl.when(dst != my)
                def _():
                    pltpu.make_async_remote_copy(
                        src_ref=y_buf.at[lax.rem(tt, 2), :, pl.ds(j * N_PER, N_PER)],
                        dst_ref=out_ref.at[pl.ds(my * M_PER, M_PER), :],
                        send_sem=send_sems.at[dst],
                        recv_sem=recv_sems.at[my],
                        device_id=(dst,),
                        device_id_type=pl.DeviceIdType.MESH,
                    ).wait_send()
        for s in range(1, N_DEV):
            src = lax.rem(my + s, N_DEV)
            pltpu.make_async_remote_copy(
                src_ref=y_buf.at[0, :, pl.ds(0, N_PER)],
                dst_ref=out_ref.at[pl.ds(src * M_PER, M_PER), :],
                send_sem=send_sems.at[src],
                recv_sem=recv_sems.at[src],
                device_id=(src,),
                device_id_type=pl.DeviceIdType.MESH,
            ).wait_recv()


def kernel(x, w_mat):
    my = lax.axis_index("i")
    sched = lax.rem(my // GRP + 1 + jnp.arange(N_GRP, dtype=jnp.int32), N_GRP)

    grid_spec = pltpu.PrefetchScalarGridSpec(
        num_scalar_prefetch=1,
        grid=(N_GRP,),
        in_specs=[
            pl.BlockSpec((M_PER, K), lambda t, s: (0, 0)),
            pl.BlockSpec((K, GW), lambda t, s: (0, s[t])),
        ],
        out_specs=pl.BlockSpec((N_DEV * M_PER, N_PER), lambda t, s: (0, 0)),
        scratch_shapes=[
            pltpu.VMEM((2, M_PER, GW), jnp.float32),
            pltpu.SemaphoreType.DMA((N_DEV,)),
            pltpu.SemaphoreType.DMA((N_DEV,)),
        ],
    )
    return pl.pallas_call(
        _body,
        grid_spec=grid_spec,
        out_shape=jax.ShapeDtypeStruct((N_DEV * M_PER, N_PER), jnp.float32),
        compiler_params=pltpu.CompilerParams(vmem_limit_bytes=60 * 1024 * 1024),
    )(sched, x, w_mat)


# device time: 70613 ns/iter; 1.8252x vs baseline; 1.2858x over previous
import jax
import jax.numpy as jnp
from jax import lax
from jax.experimental import pallas as pl
from jax.experimental.pallas import tpu as pltpu

N_DEV = 32
M_PER = 256
K = 8192
N_TOT = 4096
N_PER = 128
GRP = 4
GW = GRP * N_PER
N_GRP = N_TOT // GW


def _body(sched_ref, x_ref, w_ref, out_ref, y_buf, rbuf, send_sems, recv_sems):
    t = pl.program_id(0)
    my = lax.axis_index("i")
    g = sched_ref[t]
    slot = lax.rem(t, 2)

    @pl.when(t >= 2)
    def _():
        g_old = sched_ref[t - 2]
        for j in range(GRP):
            dst = g_old * GRP + j

            @pl.when(dst != my)
            def _():
                pltpu.make_async_remote_copy(
                    src_ref=y_buf.at[slot, :, pl.ds(j * N_PER, N_PER)],
                    dst_ref=rbuf.at[pl.ds(my * M_PER, M_PER), :],
                    send_sem=send_sems.at[dst],
                    recv_sem=recv_sems.at[my],
                    device_id=(dst,),
                    device_id_type=pl.DeviceIdType.MESH,
                ).wait_send()

    y_buf[slot] = jnp.maximum(
        jnp.dot(x_ref[...], w_ref[...], preferred_element_type=jnp.float32),
        0.0,
    ).astype(jnp.bfloat16)

    for j in range(GRP):
        dst = g * GRP + j

        @pl.when(dst != my)
        def _():
            pltpu.make_async_remote_copy(
                src_ref=y_buf.at[slot, :, pl.ds(j * N_PER, N_PER)],
                dst_ref=rbuf.at[pl.ds(my * M_PER, M_PER), :],
                send_sem=send_sems.at[dst],
                recv_sem=recv_sems.at[my],
                device_id=(dst,),
                device_id_type=pl.DeviceIdType.MESH,
            ).start()

        @pl.when(dst == my)
        def _():
            rbuf[pl.ds(my * M_PER, M_PER), :] = y_buf[
                slot, :, pl.ds(j * N_PER, N_PER)
            ]

    @pl.when(t == N_GRP - 1)
    def _():
        for tt in (N_GRP - 2, N_GRP - 1):
            g_late = sched_ref[tt]
            for j in range(GRP):
                dst = g_late * GRP + j

                @pl.when(dst != my)
                def _():
                    pltpu.make_async_remote_copy(
                        src_ref=y_buf.at[lax.rem(tt, 2), :, pl.ds(j * N_PER, N_PER)],
                        dst_ref=rbuf.at[pl.ds(my * M_PER, M_PER), :],
                        send_sem=send_sems.at[dst],
                        recv_sem=recv_sems.at[my],
                        device_id=(dst,),
                        device_id_type=pl.DeviceIdType.MESH,
                    ).wait_send()
        for s in range(1, N_DEV):
            src = lax.rem(my + s, N_DEV)
            pltpu.make_async_remote_copy(
                src_ref=y_buf.at[0, :, pl.ds(0, N_PER)],
                dst_ref=rbuf.at[pl.ds(src * M_PER, M_PER), :],
                send_sem=send_sems.at[src],
                recv_sem=recv_sems.at[src],
                device_id=(src,),
                device_id_type=pl.DeviceIdType.MESH,
            ).wait_recv()
        out_ref[...] = rbuf[...].astype(jnp.float32)


def kernel(x, w_mat):
    my = lax.axis_index("i")
    sched = lax.rem(my // GRP + 1 + jnp.arange(N_GRP, dtype=jnp.int32), N_GRP)

    grid_spec = pltpu.PrefetchScalarGridSpec(
        num_scalar_prefetch=1,
        grid=(N_GRP,),
        in_specs=[
            pl.BlockSpec((M_PER, K), lambda t, s: (0, 0)),
            pl.BlockSpec((K, GW), lambda t, s: (0, s[t])),
        ],
        out_specs=pl.BlockSpec((N_DEV * M_PER, N_PER), lambda t, s: (0, 0)),
        scratch_shapes=[
            pltpu.VMEM((2, M_PER, GW), jnp.bfloat16),
            pltpu.VMEM((N_DEV * M_PER, N_PER), jnp.bfloat16),
            pltpu.SemaphoreType.DMA((N_DEV,)),
            pltpu.SemaphoreType.DMA((N_DEV,)),
        ],
    )
    return pl.pallas_call(
        _body,
        grid_spec=grid_spec,
        out_shape=jax.ShapeDtypeStruct((N_DEV * M_PER, N_PER), jnp.float32),
        compiler_params=pltpu.CompilerParams(vmem_limit_bytes=60 * 1024 * 1024),
    )(sched, x, w_mat)


# device time: 68746 ns/iter; 1.8747x vs baseline; 1.0272x over previous
import jax
import jax.numpy as jnp
from jax import lax
from jax.experimental import pallas as pl
from jax.experimental.pallas import tpu as pltpu

N_DEV = 32
M_PER = 256
K = 8192
N_TOT = 4096
N_PER = 128
GRP = 2
GW = GRP * N_PER
N_GRP = N_TOT // GW


def _send_desc(y_buf, rbuf, send_sems, recv_sems, slot, j, my, dst):
    return pltpu.make_async_remote_copy(
        src_ref=y_buf.at[slot, :, pl.ds(j * N_PER, N_PER)],
        dst_ref=rbuf.at[pl.ds(my * M_PER, M_PER), :],
        send_sem=send_sems.at[dst],
        recv_sem=recv_sems.at[my],
        device_id=(dst,),
        device_id_type=pl.DeviceIdType.MESH,
    )


def _body(sched_ref, x_ref, w_ref, out_ref, y_buf, rbuf, send_sems, recv_sems):
    t = pl.program_id(0)
    my = lax.axis_index("i")
    g = sched_ref[t]
    slot = lax.rem(t, 2)

    @pl.when(t >= 2)
    def _():
        g_old = sched_ref[t - 2]
        for j in range(GRP):
            dst = g_old * GRP + j

            @pl.when(dst != my)
            def _():
                _send_desc(y_buf, rbuf, send_sems, recv_sems, slot, j, my, dst
                           ).wait_send()

    y_buf[slot] = jnp.maximum(
        jnp.dot(x_ref[...], w_ref[...], preferred_element_type=jnp.float32),
        0.0,
    ).astype(jnp.bfloat16)

    for j in range(GRP):
        dst = g * GRP + j

        @pl.when(dst != my)
        def _():
            _send_desc(y_buf, rbuf, send_sems, recv_sems, slot, j, my, dst
                       ).start()

        @pl.when(dst == my)
        def _():
            rbuf[pl.ds(my * M_PER, M_PER), :] = y_buf[
                slot, :, pl.ds(j * N_PER, N_PER)
            ]

    @pl.when(t == N_GRP - 1)
    def _():
        for tt in (N_GRP - 2, N_GRP - 1):
            g_late = sched_ref[tt]
            for j in range(GRP):
                dst = g_late * GRP + j

                @pl.when(dst != my)
                def _():
                    _send_desc(y_buf, rbuf, send_sems, recv_sems,
                               lax.rem(tt, 2), j, my, dst).wait_send()

        q = my // GRP
        for tt in range(N_GRP):
            ps = lax.rem(q + 2 * N_GRP - 1 - tt, N_GRP)
            for j in range(GRP):
                src = ps * GRP + j

                @pl.when(src != my)
                def _():
                    pltpu.make_async_remote_copy(
                        src_ref=y_buf.at[0, :, pl.ds(0, N_PER)],
                        dst_ref=rbuf.at[pl.ds(src * M_PER, M_PER), :],
                        send_sem=send_sems.at[src],
                        recv_sem=recv_sems.at[src],
                        device_id=(src,),
                        device_id_type=pl.DeviceIdType.MESH,
                    ).wait_recv()
            rows = GRP * M_PER
            out_ref[pl.ds(ps * rows, rows), :] = rbuf[
                pl.ds(ps * rows, rows), :
            ].astype(jnp.float32)


def kernel(x, w_mat):
    my = lax.axis_index("i")
    sched = lax.rem(my // GRP + 1 + jnp.arange(N_GRP, dtype=jnp.int32), N_GRP)

    grid_spec = pltpu.PrefetchScalarGridSpec(
        num_scalar_prefetch=1,
        grid=(N_GRP,),
        in_specs=[
            pl.BlockSpec((M_PER, K), lambda t, s: (0, 0)),
            pl.BlockSpec((K, GW), lambda t, s: (0, s[t])),
        ],
        out_specs=pl.BlockSpec((N_DEV * M_PER, N_PER), lambda t, s: (0, 0)),
        scratch_shapes=[
            pltpu.VMEM((2, M_PER, GW), jnp.bfloat16),
            pltpu.VMEM((N_DEV * M_PER, N_PER), jnp.bfloat16),
            pltpu.SemaphoreType.DMA((N_DEV,)),
            pltpu.SemaphoreType.DMA((N_DEV,)),
        ],
    )
    return pl.pallas_call(
        _body,
        grid_spec=grid_spec,
        out_shape=jax.ShapeDtypeStruct((N_DEV * M_PER, N_PER), jnp.float32),
        compiler_params=pltpu.CompilerParams(vmem_limit_bytes=60 * 1024 * 1024),
    )(sched, x, w_mat)


# device time: 67767 ns/iter; 1.9018x vs baseline; 1.0144x over previous
import jax
import jax.numpy as jnp
from jax import lax
from jax.experimental import pallas as pl
from jax.experimental.pallas import tpu as pltpu

N_DEV = 32
M_PER = 256
K = 8192
N_TOT = 4096
N_PER = 128
GRP = 2
GW = GRP * N_PER
N_GRP = N_TOT // GW


def _send_desc(y_buf, rbuf, send_sems, recv_sems, slot, j, my, dst):
    return pltpu.make_async_remote_copy(
        src_ref=y_buf.at[slot, :, pl.ds(j * N_PER, N_PER)],
        dst_ref=rbuf.at[pl.ds(my * M_PER, M_PER), :],
        send_sem=send_sems.at[dst],
        recv_sem=recv_sems.at[my],
        device_id=(dst,),
        device_id_type=pl.DeviceIdType.MESH,
    )


def _body(sched_ref, x_ref, w_ref, out_ref, y_buf, rbuf, send_sems, recv_sems):
    t = pl.program_id(0)
    my = lax.axis_index("i")
    g = sched_ref[t]
    slot = lax.rem(t, 4)

    @pl.when(t >= 4)
    def _():
        g_old = sched_ref[t - 4]
        for j in range(GRP):
            dst = g_old * GRP + j

            @pl.when(dst != my)
            def _():
                _send_desc(y_buf, rbuf, send_sems, recv_sems, slot, j, my, dst
                           ).wait_send()

    y_buf[slot] = jnp.maximum(
        jnp.dot(x_ref[...], w_ref[...], preferred_element_type=jnp.float32),
        0.0,
    ).astype(jnp.bfloat16)

    for j in range(GRP):
        dst = g * GRP + j

        @pl.when(dst != my)
        def _():
            _send_desc(y_buf, rbuf, send_sems, recv_sems, slot, j, my, dst
                       ).start()

        @pl.when(dst == my)
        def _():
            rbuf[pl.ds(my * M_PER, M_PER), :] = y_buf[
                slot, :, pl.ds(j * N_PER, N_PER)
            ]

    @pl.when(t == N_GRP - 1)
    def _():
        q = my // GRP
        for tt in range(N_GRP):
            ps = lax.rem(q + 2 * N_GRP - 1 - tt, N_GRP)
            for j in range(GRP):
                src = ps * GRP + j

                @pl.when(src != my)
                def _():
                    pltpu.make_async_remote_copy(
                        src_ref=y_buf.at[0, :, pl.ds(0, N_PER)],
                        dst_ref=rbuf.at[pl.ds(src * M_PER, M_PER), :],
                        send_sem=send_sems.at[src],
                        recv_sem=recv_sems.at[src],
                        device_id=(src,),
                        device_id_type=pl.DeviceIdType.MESH,
                    ).wait_recv()
            rows = GRP * M_PER
            out_ref[pl.ds(ps * rows, rows), :] = rbuf[
                pl.ds(ps * rows, rows), :
            ].astype(jnp.float32)

        for tt in range(N_GRP - 4, N_GRP):
            g_late = sched_ref[tt]
            for j in range(GRP):
                dst = g_late * GRP + j

                @pl.when(dst != my)
                def _():
                    _send_desc(y_buf, rbuf, send_sems, recv_sems,
                               lax.rem(tt, 4), j, my, dst).wait_send()


def kernel(x, w_mat):
    my = lax.axis_index("i")
    sched = lax.rem(my // GRP + 1 + jnp.arange(N_GRP, dtype=jnp.int32), N_GRP)

    grid_spec = pltpu.PrefetchScalarGridSpec(
        num_scalar_prefetch=1,
        grid=(N_GRP,),
        in_specs=[
            pl.BlockSpec((M_PER, K), lambda t, s: (0, 0)),
            pl.BlockSpec((K, GW), lambda t, s: (0, s[t])),
        ],
        out_specs=pl.BlockSpec((N_DEV * M_PER, N_PER), lambda t, s: (0, 0)),
        scratch_shapes=[
            pltpu.VMEM((4, M_PER, GW), jnp.bfloat16),
            pltpu.VMEM((N_DEV * M_PER, N_PER), jnp.bfloat16),
            pltpu.SemaphoreType.DMA((N_DEV,)),
            pltpu.SemaphoreType.DMA((N_DEV,)),
        ],
    )
    return pl.pallas_call(
        _body,
        grid_spec=grid_spec,
        out_shape=jax.ShapeDtypeStruct((N_DEV * M_PER, N_PER), jnp.float32),
        compiler_params=pltpu.CompilerParams(vmem_limit_bytes=60 * 1024 * 1024),
    )(sched, x, w_mat)


# device time: 63801 ns/iter; 2.0200x vs baseline; 1.0622x over previous
import jax
import jax.numpy as jnp
from jax import lax
from jax.experimental import pallas as pl
from jax.experimental.pallas import tpu as pltpu

N_DEV = 32
M_PER = 256
K = 8192
N_TOT = 4096
N_PER = 128
GRP = 2
GW = GRP * N_PER
N_GRP = N_TOT // GW


def _send_desc(y_buf, rbuf, send_sems, recv_sems, slot, j, my, dst):
    return pltpu.make_async_remote_copy(
        src_ref=y_buf.at[slot, :, pl.ds(j * N_PER, N_PER)],
        dst_ref=rbuf.at[pl.ds(my * M_PER, M_PER), :],
        send_sem=send_sems.at[dst],
        recv_sem=recv_sems.at[my],
        device_id=(dst,),
        device_id_type=pl.DeviceIdType.MESH,
    )


def _body(sched_ref, x_ref, w_ref, out_ref, y_buf, rbuf, send_sems, recv_sems):
    t = pl.program_id(0)
    my = lax.axis_index("i")
    g = sched_ref[t]
    slot = lax.rem(t, 4)

    barrier_sem = pltpu.get_barrier_semaphore()

    @pl.when(t == 0)
    def _():
        for p in range(N_DEV):
            @pl.when(p != my)
            def _():
                pl.semaphore_signal(
                    barrier_sem, inc=1,
                    device_id=(p,), device_id_type=pl.DeviceIdType.MESH,
                )

    @pl.when(t >= 4)
    def _():
        g_old = sched_ref[t - 4]
        for j in range(GRP):
            dst = g_old * GRP + j

            @pl.when(dst != my)
            def _():
                _send_desc(y_buf, rbuf, send_sems, recv_sems, slot, j, my, dst
                           ).wait_send()

    y_buf[slot] = jnp.maximum(
        jnp.dot(x_ref[...], w_ref[...], preferred_element_type=jnp.float32),
        0.0,
    ).astype(jnp.bfloat16)

    @pl.when(t == 0)
    def _():
        pl.semaphore_wait(barrier_sem, N_DEV - 1)

    for j in range(GRP):
        dst = g * GRP + j

        @pl.when(dst != my)
        def _():
            _send_desc(y_buf, rbuf, send_sems, recv_sems, slot, j, my, dst
                       ).start()

        @pl.when(dst == my)
        def _():
            rbuf[pl.ds(my * M_PER, M_PER), :] = y_buf[
                slot, :, pl.ds(j * N_PER, N_PER)
            ]

    @pl.when(t == N_GRP - 1)
    def _():
        q = my // GRP
        for tt in range(N_GRP):
            ps = lax.rem(q + 2 * N_GRP - 1 - tt, N_GRP)
            for j in range(GRP):
                src = ps * GRP + j

                @pl.when(src != my)
                def _():
                    pltpu.make_async_remote_copy(
                        src_ref=y_buf.at[0, :, pl.ds(0, N_PER)],
                        dst_ref=rbuf.at[pl.ds(src * M_PER, M_PER), :],
                        send_sem=send_sems.at[src],
                        recv_sem=recv_sems.at[src],
                        device_id=(src,),
                        device_id_type=pl.DeviceIdType.MESH,
                    ).wait_recv()
            rows = GRP * M_PER
            out_ref[pl.ds(ps * rows, rows), :] = rbuf[
                pl.ds(ps * rows, rows), :
            ].astype(jnp.float32)

        for tt in range(N_GRP - 4, N_GRP):
            g_late = sched_ref[tt]
            for j in range(GRP):
                dst = g_late * GRP + j

                @pl.when(dst != my)
                def _():
                    _send_desc(y_buf, rbuf, send_sems, recv_sems,
                               lax.rem(tt, 4), j, my, dst).wait_send()


def kernel(x, w_mat):
    my = lax.axis_index("i")
    sched = lax.rem(my // GRP + 1 + jnp.arange(N_GRP, dtype=jnp.int32), N_GRP)

    grid_spec = pltpu.PrefetchScalarGridSpec(
        num_scalar_prefetch=1,
        grid=(N_GRP,),
        in_specs=[
            pl.BlockSpec((M_PER, K), lambda t, s: (0, 0)),
            pl.BlockSpec((K, GW), lambda t, s: (0, s[t])),
        ],
        out_specs=pl.BlockSpec((N_DEV * M_PER, N_PER), lambda t, s: (0, 0)),
        scratch_shapes=[
            pltpu.VMEM((4, M_PER, GW), jnp.bfloat16),
            pltpu.VMEM((N_DEV * M_PER, N_PER), jnp.bfloat16),
            pltpu.SemaphoreType.DMA((N_DEV,)),
            pltpu.SemaphoreType.DMA((N_DEV,)),
        ],
    )
    return pl.pallas_call(
        _body,
        grid_spec=grid_spec,
        out_shape=jax.ShapeDtypeStruct((N_DEV * M_PER, N_PER), jnp.float32),
        compiler_params=pltpu.CompilerParams(
            vmem_limit_bytes=60 * 1024 * 1024, collective_id=0,
        ),
    )(sched, x, w_mat)
